# Initial kernel scaffold; baseline (speedup 1.0000x reference)
import jax
import jax.numpy as jnp
from jax.experimental import pallas as pl


def kernel(x, edge_index, W1, b1, W2, b2, W3, b3):
    raise NotImplementedError("write your pallas kernel here")


# speedup vs baseline: 30.5279x; 30.5279x over previous
import functools

import jax
import jax.numpy as jnp
from jax import lax
from jax.experimental import pallas as pl
from jax.experimental.pallas import tpu as pltpu
from jax.experimental.pallas import tpu_sc as plsc

N = 10000
E = 320000
NPAD = 10112
CHUNK = 128
CPT = 79
EPT = CPT * CHUNK
TILES = 32
EPAD = TILES * EPT
DEGW = 16
ROWS_PER_TILE = NPAD // 16

_MESH = dict(core_axis_name="c", subcore_axis_name="s")


@functools.partial(
    pl.kernel,
    out_type=jax.ShapeDtypeStruct((2, NPAD, DEGW), jnp.float32),
    mesh=plsc.VectorSubcoreMesh(**_MESH),
    scratch_types=[
        pltpu.VMEM((CPT, CHUNK), jnp.int32),
        pltpu.VMEM((CHUNK, DEGW), jnp.float32),
        pltpu.VMEM_SHARED((NPAD, DEGW), jnp.float32),
    ],
    compiler_params=pltpu.CompilerParams(use_tc_tiling_on_sc=False),
)
def _deg_kernel(dst_hbm, ones_hbm, zeros_hbm, out_hbm, dst_v, ones_v, acc_sh):
    cid = lax.axis_index("c")
    sid = lax.axis_index("s")
    wid = cid * 16 + sid
    pltpu.sync_copy(dst_hbm.at[wid], dst_v)
    pltpu.sync_copy(ones_hbm, ones_v)
    row0 = sid * ROWS_PER_TILE
    pltpu.sync_copy(zeros_hbm.at[pl.ds(row0, ROWS_PER_TILE)],
                    acc_sh.at[pl.ds(row0, ROWS_PER_TILE)])
    plsc.subcore_barrier()

    def body(j, carry):
        pltpu.sync_copy(ones_v, acc_sh.at[dst_v.at[j]], add=True)
        return carry

    lax.fori_loop(0, CPT, body, 0)
    plsc.subcore_barrier()
    pltpu.sync_copy(acc_sh.at[pl.ds(row0, ROWS_PER_TILE)],
                    out_hbm.at[cid, pl.ds(row0, ROWS_PER_TILE)])


def _make_scatter(D):
    @functools.partial(
        pl.kernel,
        out_type=jax.ShapeDtypeStruct((2, NPAD, D), jnp.float32),
        mesh=plsc.VectorSubcoreMesh(**_MESH),
        scratch_types=[
            pltpu.VMEM((CPT, CHUNK), jnp.int32),
            pltpu.VMEM((CPT, CHUNK), jnp.int32),
            pltpu.VMEM((CHUNK, D), jnp.float32),
            pltpu.VMEM_SHARED((NPAD, D), jnp.float32),
            pltpu.SemaphoreType.DMA,
        ],
        compiler_params=pltpu.CompilerParams(use_tc_tiling_on_sc=False),
    )
    def _scat(src_hbm, dst_hbm, y_hbm, zeros_hbm, out_hbm,
              src_v, dst_v, buf, acc_sh, sem):
        cid = lax.axis_index("c")
        sid = lax.axis_index("s")
        wid = cid * 16 + sid
        pltpu.sync_copy(src_hbm.at[wid], src_v)
        pltpu.sync_copy(dst_hbm.at[wid], dst_v)
        row0 = sid * ROWS_PER_TILE
        pltpu.sync_copy(zeros_hbm.at[pl.ds(row0, ROWS_PER_TILE)],
                        acc_sh.at[pl.ds(row0, ROWS_PER_TILE)])
        plsc.subcore_barrier()

        def body(j, carry):
            pltpu.async_copy(y_hbm.at[src_v.at[j]], buf, sem).wait()
            pltpu.sync_copy(buf, acc_sh.at[dst_v.at[j]], add=True)
            return carry

        lax.fori_loop(0, CPT, body, 0)
        plsc.subcore_barrier()
        pltpu.sync_copy(acc_sh.at[pl.ds(row0, ROWS_PER_TILE)],
                        out_hbm.at[cid, pl.ds(row0, ROWS_PER_TILE)])

    return _scat


_scat32 = _make_scatter(32)
_scat16 = _make_scatter(16)


_RB = 2000


def _dis_block(degp_ref):
    deg = degp_ref[0][:, 0:1] + degp_ref[1][:, 0:1] + 1.0
    return lax.rsqrt(deg)


def _tcA_body(x_ref, w_ref, degp_ref, y_ref):
    xw = jnp.dot(x_ref[...], w_ref[...], preferred_element_type=jnp.float32)
    y_ref[...] = xw * _dis_block(degp_ref)


_tcA = pl.pallas_call(
    _tcA_body,
    grid=(N // _RB,),
    in_specs=[
        pl.BlockSpec((_RB, 128), lambda i: (i, 0)),
        pl.BlockSpec((128, 32), lambda i: (0, 0)),
        pl.BlockSpec((2, _RB, DEGW), lambda i: (0, i, 0)),
    ],
    out_specs=pl.BlockSpec((_RB, 32), lambda i: (i, 0)),
    out_shape=jax.ShapeDtypeStruct((N, 32), jnp.float32),
)


def _tcB_body(s_ref, y1_ref, degp_ref, b1_ref, w2_ref, y2_ref):
    dis = _dis_block(degp_ref)
    h = dis * (s_ref[0] + s_ref[1] + y1_ref[...]) + b1_ref[...]
    h = jnp.maximum(h, 0.0)
    y2_ref[...] = jnp.dot(h, w2_ref[...],
                          preferred_element_type=jnp.float32) * dis


_tcB = pl.pallas_call(
    _tcB_body,
    grid=(N // _RB,),
    in_specs=[
        pl.BlockSpec((2, _RB, 32), lambda i: (0, i, 0)),
        pl.BlockSpec((_RB, 32), lambda i: (i, 0)),
        pl.BlockSpec((2, _RB, DEGW), lambda i: (0, i, 0)),
        pl.BlockSpec((1, 32), lambda i: (0, 0)),
        pl.BlockSpec((32, 16), lambda i: (0, 0)),
    ],
    out_specs=pl.BlockSpec((_RB, 16), lambda i: (i, 0)),
    out_shape=jax.ShapeDtypeStruct((N, 16), jnp.float32),
)


def _tcC_body(s_ref, y2_ref, degp_ref, b2_ref, w3_ref, b3_ref, out_ref):
    dis = _dis_block(degp_ref)
    h = dis * (s_ref[0] + s_ref[1] + y2_ref[...]) + b2_ref[...]
    h = jnp.maximum(h, 0.0)
    out_ref[...] = jnp.dot(h, w3_ref[...],
                           preferred_element_type=jnp.float32) + b3_ref[...]


_tcC = pl.pallas_call(
    _tcC_body,
    grid=(N // _RB,),
    in_specs=[
        pl.BlockSpec((2, _RB, 16), lambda i: (0, i, 0)),
        pl.BlockSpec((_RB, 16), lambda i: (i, 0)),
        pl.BlockSpec((2, _RB, DEGW), lambda i: (0, i, 0)),
        pl.BlockSpec((1, 16), lambda i: (0, 0)),
        pl.BlockSpec((16, 2), lambda i: (0, 0)),
        pl.BlockSpec((1, 2), lambda i: (0, 0)),
    ],
    out_specs=pl.BlockSpec((_RB, 2), lambda i: (i, 0)),
    out_shape=jax.ShapeDtypeStruct((N, 2), jnp.float32),
)


def kernel(x, edge_index, W1, b1, W2, b2, W3, b3):
    src = edge_index[0].astype(jnp.int32)
    dst = edge_index[1].astype(jnp.int32)
    src_p = jnp.concatenate(
        [src, jnp.zeros((EPAD - E,), jnp.int32)]).reshape(TILES, CPT, CHUNK)
    dst_p = jnp.concatenate(
        [dst, jnp.full((EPAD - E,), N, jnp.int32)]).reshape(TILES, CPT, CHUNK)

    ones_deg = jnp.ones((CHUNK, DEGW), jnp.float32)
    zeros_deg = jnp.zeros((NPAD, DEGW), jnp.float32)
    zeros32 = jnp.zeros((NPAD, 32), jnp.float32)
    zeros16 = jnp.zeros((NPAD, 16), jnp.float32)

    degp = _deg_kernel(dst_p, ones_deg, zeros_deg)
    y1 = _tcA(x, W1, degp)
    s1 = _scat32(src_p, dst_p, y1, zeros32)
    y2 = _tcB(s1, y1, degp, b1.reshape(1, 32), W2)
    s2 = _scat16(src_p, dst_p, y2, zeros16)
    out = _tcC(s2, y2, degp, b2.reshape(1, 16), W3, b3.reshape(1, 2))
    return out
